# baseline (speedup 1.0000x reference)
import functools
import jax, jax.numpy as jnp
from jax import lax
from jax.experimental import pallas as pl
from jax.experimental.pallas import tpu as pltpu

N_TEXT_C = 9000
N_NONTEXT_C = 1000
E_C = 160000
D_C = 256
K_TOP = 4096
CAP = 5120
NCH = CAP // 128
HB = 2000


def _hist_body(u_ref, p_ref, o_ref):
    i = pl.program_id(0)
    u = u_ref[...]
    shift = p_ref[0, 0]
    pmask = p_ref[0, 1]
    pval = p_ref[0, 2]
    sel = (u & pmask) == pval
    digit = (u >> shift) & jnp.uint32(0xFF)
    iota = lax.broadcasted_iota(jnp.uint32, (1, 256), 1)
    oh = jnp.where((digit == iota) & sel, 1.0, 0.0)
    h = jnp.sum(oh, axis=0, keepdims=True)
    @pl.when(i == 0)
    def _():
        o_ref[...] = h
    @pl.when(i != 0)
    def _():
        o_ref[...] = o_ref[...] + h


def _histogram(u2d, shift, pmask, pval):
    params = jnp.stack([jnp.uint32(shift), pmask, pval]).reshape(1, 3)
    return pl.pallas_call(
        _hist_body,
        grid=(E_C // HB,),
        in_specs=[pl.BlockSpec((HB, 1), lambda i: (i, 0)),
                  pl.BlockSpec((1, 3), lambda i: (0, 0), memory_space=pltpu.SMEM)],
        out_specs=pl.BlockSpec((1, 256), lambda i: (0, 0)),
        out_shape=jax.ShapeDtypeStruct((1, 256), jnp.float32),
    )(u2d, params)[0]


def _rank_body(rk_ref, rid_ref, ck_ref, cid_ref, o_ref):
    rk = rk_ref[...]
    rid = rid_ref[...]
    acc = jnp.zeros((128, 1), jnp.int32)
    for cb in range(NCH):
        ck = ck_ref[cb, :].reshape(1, 128)
        cid = cid_ref[cb, :].reshape(1, 128)
        gt = ck > rk
        tie = (ck == rk) & (cid < rid)
        acc = acc + jnp.sum((gt | tie).astype(jnp.int32), axis=1, keepdims=True)
    o_ref[...] = acc


def _ranks(sel_keys_col, sel_ids_col, sel_keys_row, sel_ids_row):
    return pl.pallas_call(
        _rank_body,
        grid=(NCH,),
        in_specs=[pl.BlockSpec((128, 1), lambda i: (i, 0)),
                  pl.BlockSpec((128, 1), lambda i: (i, 0)),
                  pl.BlockSpec((NCH, 128), lambda i: (0, 0)),
                  pl.BlockSpec((NCH, 128), lambda i: (0, 0))],
        out_specs=pl.BlockSpec((128, 1), lambda i: (i, 0)),
        out_shape=jax.ShapeDtypeStruct((CAP, 1), jnp.int32),
    )(sel_keys_col, sel_ids_col, sel_keys_row, sel_ids_row)


def _perm_body(ranks_ref, idsc_ref, o_ref):
    j0 = pl.program_id(0) * 512
    iota = lax.broadcasted_iota(jnp.int32, (1, 512), 1) + j0
    acc = jnp.zeros((1, 512), jnp.int32)
    for cb in range(NCH):
        rk = ranks_ref[cb * 128:(cb + 1) * 128, :]
        idc = idsc_ref[cb * 128:(cb + 1) * 128, :]
        sel = jnp.where(rk == iota, idc, 0)
        acc = acc + jnp.sum(sel, axis=0, keepdims=True)
    o_ref[...] = acc


def _permute(ranks_col, sel_ids_col):
    return pl.pallas_call(
        _perm_body,
        grid=(K_TOP // 512,),
        in_specs=[pl.BlockSpec((CAP, 1), lambda i: (0, 0)),
                  pl.BlockSpec((CAP, 1), lambda i: (0, 0))],
        out_specs=pl.BlockSpec((1, 512), lambda i: (0, i)),
        out_shape=jax.ShapeDtypeStruct((1, K_TOP), jnp.int32),
    )(ranks_col, sel_ids_col)


def _topk_pallas(pred_flat):
    b = lax.bitcast_convert_type(pred_flat, jnp.int32)
    s_key = jnp.where(b < 0, jnp.int32(0x7FFFFFFF) ^ b, b)
    u_key = lax.bitcast_convert_type(s_key, jnp.uint32) ^ jnp.uint32(0x80000000)
    u2d = u_key.reshape(E_C, 1)

    kneed = jnp.float32(K_TOP)
    pmask = jnp.uint32(0)
    pval = jnp.uint32(0)
    tie_h = jnp.float32(0)
    for shift in (24, 16, 8, 0):
        h = _histogram(u2d, shift, pmask, pval)
        cge = jnp.cumsum(h[::-1])[::-1]
        ok = cge >= kneed
        bsel = jnp.sum(ok.astype(jnp.int32)) - 1
        above = cge[bsel] - h[bsel]
        kneed = kneed - above
        tie_h = h[bsel]
        pval = pval | (bsel.astype(jnp.uint32) << shift)
        pmask = pmask | (jnp.uint32(0xFF) << shift)
    thr = pval
    count_ge = (jnp.float32(K_TOP) - kneed + tie_h).astype(jnp.int32)

    sel_idx = jnp.where(u_key >= thr, size=CAP, fill_value=0)[0]
    jpos = jnp.arange(CAP, dtype=jnp.int32)
    valid = jpos < count_ge
    sel_keys = jnp.where(valid, s_key[sel_idx], jnp.int32(-0x80000000))
    sel_ids = jnp.where(valid, sel_idx.astype(jnp.int32), jnp.int32(-1))

    ranks = _ranks(sel_keys.reshape(CAP, 1), sel_ids.reshape(CAP, 1),
                   sel_keys.reshape(NCH, 128), sel_ids.reshape(NCH, 128))
    ids_out = _permute(ranks, sel_ids.reshape(CAP, 1))
    return ids_out.reshape(K_TOP)


def kernel(h_id_tensor, t_id_tensor, r_id_tensor, q_id_tensor,
           num_non_text_entities, q_emb, entity_embs, relation_embs,
           non_text_emb, W1, b1, W2, b2):
    h_id, t_id, r_id, q_id = h_id_tensor, t_id_tensor, r_id_tensor, q_id_tensor
    n_total = N_TEXT_C + N_NONTEXT_C
    nnt_delta = jnp.asarray(num_non_text_entities, jnp.float32) - jnp.float32(N_NONTEXT_C)
    mask = jnp.zeros((n_total,), jnp.float32).at[q_id].set(1.0)
    topic = jax.nn.one_hot(mask.astype(jnp.int32), 2, dtype=jnp.float32)
    h_e = jnp.concatenate([entity_embs,
                           jnp.broadcast_to(non_text_emb, (N_NONTEXT_C, D_C))], axis=0)

    m = mask
    ones_e = jnp.ones((E_C,), jnp.float32)
    seg = jnp.concatenate([t_id, h_id + n_total,
                           t_id + 2 * n_total, h_id + 3 * n_total])
    upd = jnp.concatenate([ones_e, ones_e, m[h_id], m[t_id]])
    S = jax.ops.segment_sum(upd, seg, num_segments=4 * n_total)
    c_fwd = S[:n_total]
    c_rev = S[n_total:2 * n_total]
    s_fwd = S[2 * n_total:3 * n_total]
    s_rev = S[3 * n_total:]
    den_fwd = jnp.maximum(c_fwd, 1.0)[:, None]
    den_rev = jnp.maximum(c_rev, 1.0)[:, None]
    p1_fwd = jnp.stack([c_fwd - s_fwd, s_fwd], axis=1) / den_fwd
    p1_rev = jnp.stack([c_rev - s_rev, s_rev], axis=1) / den_rev
    p2_fwd = jax.ops.segment_sum(p1_fwd[h_id], t_id, num_segments=n_total) / den_fwd
    p2_rev = jax.ops.segment_sum(p1_rev[t_id], h_id, num_segments=n_total) / den_rev
    feats = [h_e, topic, p1_fwd, p2_fwd, p1_rev, p2_rev]
    h_full_bf = jnp.concatenate(feats, axis=1).astype(jnp.bfloat16)
    h_q = jnp.broadcast_to(q_emb.astype(jnp.bfloat16)[None, :], (E_C, D_C))
    h_r = relation_embs.astype(jnp.bfloat16)[r_id]
    h_triple = jnp.concatenate([h_q, h_full_bf[h_id], h_r, h_full_bf[t_id]], axis=1)
    hidden = jax.lax.dot_general(h_triple, W1, (((1,), (0,)), ((), ())),
                                 preferred_element_type=jnp.float32)
    hidden = jnp.maximum(hidden + b1, 0.0).astype(jnp.bfloat16)
    pred = jax.lax.dot_general(hidden, W2, (((1,), (0,)), ((), ())),
                               preferred_element_type=jnp.float32)
    pred = pred + b2 + nnt_delta
    edge_ids = _topk_pallas(pred.reshape(E_C))
    return pred, edge_ids


# speedup vs baseline: 1.1094x; 1.1094x over previous
import functools
import jax, jax.numpy as jnp
from jax import lax
from jax.experimental import pallas as pl
from jax.experimental.pallas import tpu as pltpu

N_TEXT_C = 9000
N_NONTEXT_C = 1000
E_C = 160000
D_C = 256
K_TOP = 4096
CAP = 5120
NCH = CAP // 128
HB = 2000


def _hist_body(u_ref, p_ref, o_ref):
    i = pl.program_id(0)
    u = u_ref[...]
    shift = p_ref[0, 0]
    pmask = p_ref[0, 1]
    pval = p_ref[0, 2]
    sel = (u & pmask) == pval
    digit = (u >> shift) & jnp.uint32(0xFF)
    iota = lax.broadcasted_iota(jnp.uint32, (1, 256), 1)
    oh = jnp.where((digit == iota) & sel, 1.0, 0.0)
    h = jnp.sum(oh, axis=0, keepdims=True)
    @pl.when(i == 0)
    def _():
        o_ref[...] = h
    @pl.when(i != 0)
    def _():
        o_ref[...] = o_ref[...] + h


def _histogram(u2d, shift, pmask, pval):
    params = jnp.stack([jnp.uint32(shift), pmask, pval]).reshape(1, 3)
    return pl.pallas_call(
        _hist_body,
        grid=(E_C // HB,),
        in_specs=[pl.BlockSpec((HB, 1), lambda i: (i, 0)),
                  pl.BlockSpec((1, 3), lambda i: (0, 0), memory_space=pltpu.SMEM)],
        out_specs=pl.BlockSpec((1, 256), lambda i: (0, 0)),
        out_shape=jax.ShapeDtypeStruct((1, 256), jnp.float32),
    )(u2d, params)[0]


def _rank_body(rk_ref, rid_ref, ck_ref, cid_ref, o_ref):
    rk = rk_ref[...]
    rid = rid_ref[...]
    acc = jnp.zeros((128, 1), jnp.int32)
    for cb in range(NCH):
        ck = ck_ref[cb, :].reshape(1, 128)
        cid = cid_ref[cb, :].reshape(1, 128)
        gt = ck > rk
        tie = (ck == rk) & (cid < rid)
        acc = acc + jnp.sum((gt | tie).astype(jnp.int32), axis=1, keepdims=True)
    o_ref[...] = acc


def _ranks(sel_keys_col, sel_ids_col, sel_keys_row, sel_ids_row):
    return pl.pallas_call(
        _rank_body,
        grid=(NCH,),
        in_specs=[pl.BlockSpec((128, 1), lambda i: (i, 0)),
                  pl.BlockSpec((128, 1), lambda i: (i, 0)),
                  pl.BlockSpec((NCH, 128), lambda i: (0, 0)),
                  pl.BlockSpec((NCH, 128), lambda i: (0, 0))],
        out_specs=pl.BlockSpec((128, 1), lambda i: (i, 0)),
        out_shape=jax.ShapeDtypeStruct((CAP, 1), jnp.int32),
    )(sel_keys_col, sel_ids_col, sel_keys_row, sel_ids_row)


def _perm_body(ranks_ref, idsc_ref, o_ref):
    j0 = pl.program_id(0) * 512
    iota = lax.broadcasted_iota(jnp.int32, (1, 512), 1) + j0
    acc = jnp.zeros((1, 512), jnp.int32)
    for cb in range(NCH):
        rk = ranks_ref[cb * 128:(cb + 1) * 128, :]
        idc = idsc_ref[cb * 128:(cb + 1) * 128, :]
        sel = jnp.where(rk == iota, idc, 0)
        acc = acc + jnp.sum(sel, axis=0, keepdims=True)
    o_ref[...] = acc


def _permute(ranks_col, sel_ids_col):
    return pl.pallas_call(
        _perm_body,
        grid=(K_TOP // 512,),
        in_specs=[pl.BlockSpec((CAP, 1), lambda i: (0, 0)),
                  pl.BlockSpec((CAP, 1), lambda i: (0, 0))],
        out_specs=pl.BlockSpec((1, 512), lambda i: (0, i)),
        out_shape=jax.ShapeDtypeStruct((1, K_TOP), jnp.int32),
    )(ranks_col, sel_ids_col)


def _topk_pallas(pred_flat):
    b = lax.bitcast_convert_type(pred_flat, jnp.int32)
    s_key = jnp.where(b < 0, jnp.int32(0x7FFFFFFF) ^ b, b)
    u_key = lax.bitcast_convert_type(s_key, jnp.uint32) ^ jnp.uint32(0x80000000)
    u2d = u_key.reshape(E_C, 1)

    kneed = jnp.float32(K_TOP)
    pmask = jnp.uint32(0)
    pval = jnp.uint32(0)
    tie_h = jnp.float32(0)
    for shift in (24, 16, 8, 0):
        h = _histogram(u2d, shift, pmask, pval)
        cge = jnp.cumsum(h[::-1])[::-1]
        ok = cge >= kneed
        bsel = jnp.sum(ok.astype(jnp.int32)) - 1
        above = cge[bsel] - h[bsel]
        kneed = kneed - above
        tie_h = h[bsel]
        pval = pval | (bsel.astype(jnp.uint32) << shift)
        pmask = pmask | (jnp.uint32(0xFF) << shift)
    thr = pval
    count_ge = (jnp.float32(K_TOP) - kneed + tie_h).astype(jnp.int32)

    sel_idx = jnp.where(u_key >= thr, size=CAP, fill_value=0)[0]
    jpos = jnp.arange(CAP, dtype=jnp.int32)
    valid = jpos < count_ge
    sel_keys = jnp.where(valid, s_key[sel_idx], jnp.int32(-0x80000000))
    sel_ids = jnp.where(valid, sel_idx.astype(jnp.int32), jnp.int32(-1))

    ranks = _ranks(sel_keys.reshape(CAP, 1), sel_ids.reshape(CAP, 1),
                   sel_keys.reshape(NCH, 128), sel_ids.reshape(NCH, 128))
    ids_out = _permute(ranks, sel_ids.reshape(CAP, 1))
    return ids_out.reshape(K_TOP)


def kernel(h_id_tensor, t_id_tensor, r_id_tensor, q_id_tensor,
           num_non_text_entities, q_emb, entity_embs, relation_embs,
           non_text_emb, W1, b1, W2, b2):
    h_id, t_id, r_id, q_id = h_id_tensor, t_id_tensor, r_id_tensor, q_id_tensor
    n_total = N_TEXT_C + N_NONTEXT_C
    nnt_delta = jnp.asarray(num_non_text_entities, jnp.float32) - jnp.float32(N_NONTEXT_C)
    mask = jnp.zeros((n_total,), jnp.float32).at[q_id].set(1.0)
    topic = jax.nn.one_hot(mask.astype(jnp.int32), 2, dtype=jnp.float32)
    h_e = jnp.concatenate([entity_embs,
                           jnp.broadcast_to(non_text_emb, (N_NONTEXT_C, D_C))], axis=0)

    m = mask
    ones_e = jnp.ones((E_C,), jnp.float32)
    c_fwd = jax.ops.segment_sum(ones_e, t_id, num_segments=n_total)
    c_rev = jax.ops.segment_sum(ones_e, h_id, num_segments=n_total)
    s_fwd = jax.ops.segment_sum(m[h_id], t_id, num_segments=n_total)
    s_rev = jax.ops.segment_sum(m[t_id], h_id, num_segments=n_total)
    den_fwd = jnp.maximum(c_fwd, 1.0)[:, None]
    den_rev = jnp.maximum(c_rev, 1.0)[:, None]
    p1_fwd = jnp.stack([c_fwd - s_fwd, s_fwd], axis=1) / den_fwd
    p1_rev = jnp.stack([c_rev - s_rev, s_rev], axis=1) / den_rev
    p2_fwd = jax.ops.segment_sum(p1_fwd[h_id], t_id, num_segments=n_total) / den_fwd
    p2_rev = jax.ops.segment_sum(p1_rev[t_id], h_id, num_segments=n_total) / den_rev
    feats = [h_e, topic, p1_fwd, p2_fwd, p1_rev, p2_rev]
    h_full_bf = jnp.concatenate(feats, axis=1).astype(jnp.bfloat16)
    h_q = jnp.broadcast_to(q_emb.astype(jnp.bfloat16)[None, :], (E_C, D_C))
    h_r = relation_embs.astype(jnp.bfloat16)[r_id]
    h_triple = jnp.concatenate([h_q, h_full_bf[h_id], h_r, h_full_bf[t_id]], axis=1)
    hidden = jax.lax.dot_general(h_triple, W1, (((1,), (0,)), ((), ())),
                                 preferred_element_type=jnp.float32)
    hidden = jnp.maximum(hidden + b1, 0.0).astype(jnp.bfloat16)
    pred = jax.lax.dot_general(hidden, W2, (((1,), (0,)), ((), ())),
                               preferred_element_type=jnp.float32)
    pred = pred + b2 + nnt_delta
    edge_ids = _topk_pallas(pred.reshape(E_C))
    return pred, edge_ids


# speedup vs baseline: 1.4319x; 1.2907x over previous
import functools
import jax, jax.numpy as jnp
from jax import lax
from jax.experimental import pallas as pl
from jax.experimental.pallas import tpu as pltpu

N_TEXT_C = 9000
N_NONTEXT_C = 1000
E_C = 160000
D_C = 256
K_TOP = 4096
CAP = 5120
NCH = CAP // 128
HB = 2000


def _hist_body(u_ref, p_ref, o_ref):
    i = pl.program_id(0)
    u = u_ref[...]
    shift = p_ref[0, 0]
    pmask = p_ref[0, 1]
    pval = p_ref[0, 2]
    sel = (u & pmask) == pval
    digit = (u >> shift) & jnp.uint32(0xFF)
    iota = lax.broadcasted_iota(jnp.uint32, (1, 256), 1)
    oh = jnp.where((digit == iota) & sel, 1.0, 0.0)
    h = jnp.sum(oh, axis=0, keepdims=True)
    @pl.when(i == 0)
    def _():
        o_ref[...] = h
    @pl.when(i != 0)
    def _():
        o_ref[...] = o_ref[...] + h


def _histogram(u2d, shift, pmask, pval):
    params = jnp.stack([jnp.uint32(shift), pmask, pval]).reshape(1, 3)
    return pl.pallas_call(
        _hist_body,
        grid=(E_C // HB,),
        in_specs=[pl.BlockSpec((HB, 1), lambda i: (i, 0)),
                  pl.BlockSpec((1, 3), lambda i: (0, 0), memory_space=pltpu.SMEM)],
        out_specs=pl.BlockSpec((1, 256), lambda i: (0, 0)),
        out_shape=jax.ShapeDtypeStruct((1, 256), jnp.float32),
    )(u2d, params)[0]


def _rank_body(rk_ref, rid_ref, ck_ref, cid_ref, o_ref):
    rk = rk_ref[...]
    rid = rid_ref[...]
    acc = jnp.zeros((128, 1), jnp.int32)
    for cb in range(NCH):
        ck = ck_ref[cb, :].reshape(1, 128)
        cid = cid_ref[cb, :].reshape(1, 128)
        gt = ck > rk
        tie = (ck == rk) & (cid < rid)
        acc = acc + jnp.sum((gt | tie).astype(jnp.int32), axis=1, keepdims=True)
    o_ref[...] = acc


def _ranks(sel_keys_col, sel_ids_col, sel_keys_row, sel_ids_row):
    return pl.pallas_call(
        _rank_body,
        grid=(NCH,),
        in_specs=[pl.BlockSpec((128, 1), lambda i: (i, 0)),
                  pl.BlockSpec((128, 1), lambda i: (i, 0)),
                  pl.BlockSpec((NCH, 128), lambda i: (0, 0)),
                  pl.BlockSpec((NCH, 128), lambda i: (0, 0))],
        out_specs=pl.BlockSpec((128, 1), lambda i: (i, 0)),
        out_shape=jax.ShapeDtypeStruct((CAP, 1), jnp.int32),
    )(sel_keys_col, sel_ids_col, sel_keys_row, sel_ids_row)


def _perm_body(ranks_ref, idsc_ref, o_ref):
    j0 = pl.program_id(0) * 512
    iota = lax.broadcasted_iota(jnp.int32, (1, 512), 1) + j0
    acc = jnp.zeros((1, 512), jnp.int32)
    for cb in range(NCH):
        rk = ranks_ref[cb * 128:(cb + 1) * 128, :]
        idc = idsc_ref[cb * 128:(cb + 1) * 128, :]
        sel = jnp.where(rk == iota, idc, 0)
        acc = acc + jnp.sum(sel, axis=0, keepdims=True)
    o_ref[...] = acc


def _permute(ranks_col, sel_ids_col):
    return pl.pallas_call(
        _perm_body,
        grid=(K_TOP // 512,),
        in_specs=[pl.BlockSpec((CAP, 1), lambda i: (0, 0)),
                  pl.BlockSpec((CAP, 1), lambda i: (0, 0))],
        out_specs=pl.BlockSpec((1, 512), lambda i: (0, i)),
        out_shape=jax.ShapeDtypeStruct((1, K_TOP), jnp.int32),
    )(ranks_col, sel_ids_col)


def _topk_pallas(pred_flat):
    b = lax.bitcast_convert_type(pred_flat, jnp.int32)
    s_key = jnp.where(b < 0, jnp.int32(0x7FFFFFFF) ^ b, b)
    u_key = lax.bitcast_convert_type(s_key, jnp.uint32) ^ jnp.uint32(0x80000000)
    u2d = u_key.reshape(E_C, 1)

    kneed = jnp.float32(K_TOP)
    pmask = jnp.uint32(0)
    pval = jnp.uint32(0)
    tie_h = jnp.float32(0)
    for shift in (24, 16, 8, 0):
        h = _histogram(u2d, shift, pmask, pval)
        cge = jnp.cumsum(h[::-1])[::-1]
        ok = cge >= kneed
        bsel = jnp.sum(ok.astype(jnp.int32)) - 1
        above = cge[bsel] - h[bsel]
        kneed = kneed - above
        tie_h = h[bsel]
        pval = pval | (bsel.astype(jnp.uint32) << shift)
        pmask = pmask | (jnp.uint32(0xFF) << shift)
    thr = pval
    count_ge = (jnp.float32(K_TOP) - kneed + tie_h).astype(jnp.int32)

    sel_idx = jnp.where(u_key >= thr, size=CAP, fill_value=0)[0]
    jpos = jnp.arange(CAP, dtype=jnp.int32)
    valid = jpos < count_ge
    sel_keys = jnp.where(valid, s_key[sel_idx], jnp.int32(-0x80000000))
    sel_ids = jnp.where(valid, sel_idx.astype(jnp.int32), jnp.int32(-1))

    ranks = _ranks(sel_keys.reshape(CAP, 1), sel_ids.reshape(CAP, 1),
                   sel_keys.reshape(NCH, 128), sel_ids.reshape(NCH, 128))
    ids_out = _permute(ranks, sel_ids.reshape(CAP, 1))
    return ids_out.reshape(K_TOP)


def kernel(h_id_tensor, t_id_tensor, r_id_tensor, q_id_tensor,
           num_non_text_entities, q_emb, entity_embs, relation_embs,
           non_text_emb, W1, b1, W2, b2):
    h_id, t_id, r_id, q_id = h_id_tensor, t_id_tensor, r_id_tensor, q_id_tensor
    n_total = N_TEXT_C + N_NONTEXT_C
    nnt_delta = jnp.asarray(num_non_text_entities, jnp.float32) - jnp.float32(N_NONTEXT_C)
    mask = jnp.zeros((n_total,), jnp.float32).at[q_id].set(1.0)
    topic = jax.nn.one_hot(mask.astype(jnp.int32), 2, dtype=jnp.float32)
    h_e = jnp.concatenate([entity_embs,
                           jnp.broadcast_to(non_text_emb, (N_NONTEXT_C, D_C))], axis=0)

    def conv(src, dst, x):
        s = jax.ops.segment_sum(x[src], dst, num_segments=n_total)
        c = jax.ops.segment_sum(jnp.ones((src.shape[0],), x.dtype), dst, num_segments=n_total)
        return s / jnp.maximum(c, 1.0)[:, None]

    feats = [h_e, topic]
    hp = topic
    for _ in range(2):
        hp = conv(h_id, t_id, hp); feats.append(hp)
    hp = topic
    for _ in range(2):
        hp = conv(t_id, h_id, hp); feats.append(hp)
    h_full_bf = jnp.concatenate(feats, axis=1).astype(jnp.bfloat16)
    h_q = jnp.broadcast_to(q_emb.astype(jnp.bfloat16)[None, :], (E_C, D_C))
    h_r = relation_embs.astype(jnp.bfloat16)[r_id]
    h_triple = jnp.concatenate([h_q, h_full_bf[h_id], h_r, h_full_bf[t_id]], axis=1)
    hidden = jax.lax.dot_general(h_triple, W1, (((1,), (0,)), ((), ())),
                                 preferred_element_type=jnp.float32)
    hidden = jnp.maximum(hidden + b1, 0.0).astype(jnp.bfloat16)
    pred = jax.lax.dot_general(hidden, W2, (((1,), (0,)), ((), ())),
                               preferred_element_type=jnp.float32)
    pred = pred + b2 + nnt_delta
    edge_ids = _topk_pallas(pred.reshape(E_C))
    return pred, edge_ids
